# baseline (speedup 1.0000x reference)
import functools

import jax
import jax.numpy as jnp
from jax import lax
from jax.experimental import pallas as pl
from jax.experimental.pallas import tpu as pltpu
from jax.experimental.pallas import tpu_sc as plsc

VOCAB = 1000000
EMBED_DIM = 64
NCLS = 10
CPAD = 16
BATCH = 16384
HIST = 50
PAIR = 2 * HIST + 4

R = 1024
NK = 8
STEP = NK * R
NSTEPS = VOCAB // STEP
MAIN = NSTEPS * STEP
TAIL = VOCAB - MAIN
OUTROWS = NSTEPS * R + TAIL + 1
BIASROW = (NSTEPS * R + TAIL) * 8
GROWS = OUTROWS * 8

NUM_WORKERS = 32
BAGS_PER_W = BATCH // NUM_WORKERS
PAIRS_PER_W = BAGS_PER_W // 2
IDX_PER_W = PAIRS_PER_W * PAIR
K = 16
GROUPS = PAIRS_PER_W // K


def _proj_body(*refs):
    et_refs = refs[:NK]
    et_tail_ref, w1t_ref, w2t_ref, b1_ref, b2_ref, o_ref = refs[NK:]
    i = pl.program_id(0)
    wt = lax.dot_general(w2t_ref[...], w1t_ref[...], (((0,), (1,)), ((), ())),
                         preferred_element_type=jnp.float32) * (1.0 / HIST)

    @pl.when(i < NSTEPS)
    def _main():
        ps = []
        for k in range(NK):
            ps.append(jnp.dot(wt, et_refs[k][...], preferred_element_type=jnp.float32))
        o_ref[...] = jnp.concatenate(ps, axis=0).T

    @pl.when(i == NSTEPS)
    def _tail():
        pt = jnp.dot(wt, et_tail_ref[...], preferred_element_type=jnp.float32)
        o_ref[0:TAIL, :] = jnp.concatenate([pt] * NK, axis=0).T
        o_ref[TAIL:TAIL + 1, 0:CPAD] = (
            jnp.dot(b1_ref[...], w2t_ref[...], preferred_element_type=jnp.float32)
            + b2_ref[...]
        )


def _project(et, et_tail, w1t, w2t, b1, b2):
    ins = [pl.BlockSpec((EMBED_DIM, R), (lambda k: (lambda i: (0, jnp.minimum(i * NK + k, NSTEPS * NK - 1))))(k))
           for k in range(NK)]
    ins += [pl.BlockSpec((EMBED_DIM, TAIL), lambda i: (0, 0)),
            pl.BlockSpec((EMBED_DIM, 32), lambda i: (0, 0)),
            pl.BlockSpec((32, CPAD), lambda i: (0, 0)),
            pl.BlockSpec((1, 32), lambda i: (0, 0)),
            pl.BlockSpec((1, CPAD), lambda i: (0, 0))]
    return pl.pallas_call(
        _proj_body, grid=(NSTEPS + 1,), in_specs=ins,
        out_specs=pl.BlockSpec((R, NK * CPAD), lambda i: (i, 0)),
        out_shape=jax.ShapeDtypeStruct((OUTROWS, NK * CPAD), jnp.float32),
    )(*([et] * NK), et_tail, w1t, w2t, b1, b2)


def _make_bagsum():
    mesh = plsc.VectorSubcoreMesh(core_axis_name="c", subcore_axis_name="s")

    @functools.partial(
        pl.kernel,
        mesh=mesh,
        out_type=jax.ShapeDtypeStruct((BATCH, CPAD), jnp.float32),
        scratch_types=[
            pltpu.VMEM((IDX_PER_W,), jnp.int32),
            pltpu.VMEM((2, K, 2 * HIST, CPAD), jnp.float32),
            pltpu.VMEM((BAGS_PER_W, CPAD), jnp.float32),
            pltpu.VMEM((8, CPAD), jnp.float32),
            pltpu.SemaphoreType.DMA,
        ],
        compiler_params=pltpu.CompilerParams(use_tc_tiling_on_sc=False),
    )
    def bagsum(idx_hbm, table_hbm, out_hbm, idx_v, rows_v, sums_v, bias_v, sem):
        num_cores = jax.lax.axis_size("c")
        wid = lax.axis_index("s") * num_cores + lax.axis_index("c")
        pltpu.sync_copy(idx_hbm.at[pl.ds(wid * IDX_PER_W, IDX_PER_W)], idx_v)
        pltpu.sync_copy(table_hbm.at[pl.ds(BIASROW, 8)], bias_v)

        def fire_group(g, p):
            for j in range(K):
                pltpu.async_copy(
                    table_hbm.at[idx_v.at[pl.ds((g * K + j) * PAIR, 2 * HIST)]],
                    rows_v.at[p, j], sem,
                )

        def drain_group(p):
            for j in range(K):
                pltpu.make_async_copy(
                    table_hbm.at[idx_v.at[pl.ds(0, 2 * HIST)]],
                    rows_v.at[p, j], sem,
                ).wait()

        def sum_group(g, p):
            bias = bias_v[0, :]
            for j in range(K):
                buf = rows_v.at[p, j]
                for half in range(2):
                    base = half * HIST
                    acc = buf[base, :]

                    def row_body(it, acc, _base=base, _buf=buf):
                        r = _base + 1 + it * 7
                        for u in range(7):
                            acc += _buf[r + u, :]
                        return acc

                    acc = lax.fori_loop(0, 7, row_body, acc)
                    sums_v[(g * K + j) * 2 + half, :] = acc + bias

        fire_group(0, 0)

        def outer(i, _):
            g0 = 2 * i
            fire_group(g0 + 1, 1)
            drain_group(0)
            sum_group(g0, 0)
            g2 = lax.rem(g0 + 2, GROUPS)
            fire_group(g2, 0)
            drain_group(1)
            sum_group(g0 + 1, 1)
            return 0

        lax.fori_loop(0, GROUPS // 2, outer, 0)
        drain_group(0)
        pltpu.sync_copy(sums_v, out_hbm.at[pl.ds(wid * BAGS_PER_W, BAGS_PER_W)])

    return bagsum


def kernel(text, emb_weight, fc1_w, fc1_b, fc2_w, fc2_b):
    et = emb_weight.T
    et_tail = lax.slice(et, (0, MAIN), (EMBED_DIM, VOCAB))
    w2t = jnp.pad(fc2_w.T, ((0, 0), (0, CPAD - NCLS)))
    b2p = jnp.pad(fc2_b, (0, CPAD - NCLS)).reshape(1, CPAD)
    p = _project(et, et_tail, fc1_w.T, w2t, fc1_b.reshape(1, 32), b2p)
    table16 = p.reshape(GROWS, CPAD)

    v = text.astype(jnp.int32)
    linrow = jnp.where(
        v < MAIN,
        (v & ~(STEP - 1)) | ((v & (R - 1)) << 3) | ((v >> 10) & (NK - 1)),
        MAIN + ((v - MAIN) << 3),
    )
    pairs = linrow.reshape(BATCH // 2, 2 * HIST)
    idx = jnp.pad(pairs, ((0, 0), (0, PAIR - 2 * HIST))).reshape(-1)

    sums = _make_bagsum()(idx, table16)
    return sums[:, :NCLS]


# speedup vs baseline: 1.0635x; 1.0635x over previous
import functools

import jax
import jax.numpy as jnp
from jax import lax
from jax.experimental import pallas as pl
from jax.experimental.pallas import tpu as pltpu
from jax.experimental.pallas import tpu_sc as plsc

VOCAB = 1000000
EMBED_DIM = 64
NCLS = 10
CPAD = 16
BATCH = 16384
HIST = 50
PAIR = 2 * HIST + 4

R = 1024
NK = 8
STEP = NK * R
NSTEPS = VOCAB // STEP
MAIN = NSTEPS * STEP
TAIL = VOCAB - MAIN
OUTROWS = NSTEPS * R + TAIL + 1
BIASROW = (NSTEPS * R + TAIL) * 8
GROWS = OUTROWS * 8

NUM_WORKERS = 32
BAGS_PER_W = BATCH // NUM_WORKERS
PAIRS_PER_W = BAGS_PER_W // 2
IDX_PER_W = PAIRS_PER_W * 2 * HIST
K = 8
GROUPS = PAIRS_PER_W // K


def _proj_body(*refs):
    et_refs = refs[:NK]
    et_tail_ref, w1t_ref, w2t_ref, b1_ref, b2_ref, o_ref = refs[NK:]
    i = pl.program_id(0)
    wt = lax.dot_general(w2t_ref[...], w1t_ref[...], (((0,), (1,)), ((), ())),
                         preferred_element_type=jnp.float32) * (1.0 / HIST)

    @pl.when(i < NSTEPS)
    def _main():
        ps = []
        for k in range(NK):
            ps.append(jnp.dot(wt, et_refs[k][...], preferred_element_type=jnp.float32))
        o_ref[...] = jnp.concatenate(ps, axis=0).T

    @pl.when(i == NSTEPS)
    def _tail():
        pt = jnp.dot(wt, et_tail_ref[...], preferred_element_type=jnp.float32)
        o_ref[0:TAIL, :] = jnp.concatenate([pt] * NK, axis=0).T
        o_ref[TAIL:TAIL + 1, 0:CPAD] = (
            jnp.dot(b1_ref[...], w2t_ref[...], preferred_element_type=jnp.float32)
            + b2_ref[...]
        )


def _project(et, et_tail, w1t, w2t, b1, b2):
    ins = [pl.BlockSpec((EMBED_DIM, R), (lambda k: (lambda i: (0, jnp.minimum(i * NK + k, NSTEPS * NK - 1))))(k))
           for k in range(NK)]
    ins += [pl.BlockSpec((EMBED_DIM, TAIL), lambda i: (0, 0)),
            pl.BlockSpec((EMBED_DIM, 32), lambda i: (0, 0)),
            pl.BlockSpec((32, CPAD), lambda i: (0, 0)),
            pl.BlockSpec((1, 32), lambda i: (0, 0)),
            pl.BlockSpec((1, CPAD), lambda i: (0, 0))]
    return pl.pallas_call(
        _proj_body, grid=(NSTEPS + 1,), in_specs=ins,
        out_specs=pl.BlockSpec((R, NK * CPAD), lambda i: (i, 0)),
        out_shape=jax.ShapeDtypeStruct((OUTROWS, NK * CPAD), jnp.float32),
    )(*([et] * NK), et_tail, w1t, w2t, b1, b2)


def _make_bagsum():
    mesh = plsc.VectorSubcoreMesh(core_axis_name="c", subcore_axis_name="s")

    @functools.partial(
        pl.kernel,
        mesh=mesh,
        out_type=jax.ShapeDtypeStruct((BATCH, CPAD), jnp.float32),
        scratch_types=[
            pltpu.VMEM((IDX_PER_W,), jnp.int32),
            pltpu.VMEM((2, K, PAIR, CPAD), jnp.float32),
            pltpu.VMEM((BAGS_PER_W, CPAD), jnp.float32),
            pltpu.VMEM((8, CPAD), jnp.float32),
            pltpu.SemaphoreType.DMA,
        ],
        compiler_params=pltpu.CompilerParams(use_tc_tiling_on_sc=False),
    )
    def bagsum(idx_hbm, table_hbm, out_hbm, idx_v, rows_v, sums_v, bias_v, sem):
        num_cores = jax.lax.axis_size("c")
        wid = lax.axis_index("s") * num_cores + lax.axis_index("c")
        pltpu.sync_copy(idx_hbm.at[pl.ds(wid * IDX_PER_W, IDX_PER_W)], idx_v)
        pltpu.sync_copy(table_hbm.at[pl.ds(BIASROW, 8)], bias_v)

        def _n(j):
            return 2 * HIST if j % 2 == 0 else 2 * HIST + 4

        def _off(j):
            return 0 if j % 2 == 0 else -4

        def fire_group(g, p):
            go = pl.multiple_of(g * (K * 2 * HIST), 8)
            for j in range(K):
                pltpu.async_copy(
                    table_hbm.at[idx_v.at[pl.ds(go + (2 * HIST * j + _off(j)), _n(j))]],
                    rows_v.at[p, j, pl.ds(0, _n(j))], sem,
                )

        def drain_group(p):
            for j in range(K):
                pltpu.make_async_copy(
                    table_hbm.at[idx_v.at[pl.ds(0, _n(j))]],
                    rows_v.at[p, j, pl.ds(0, _n(j))], sem,
                ).wait()

        def sum_group(g, p):
            bias = bias_v[0, :]
            for j in range(K):
                buf = rows_v.at[p, j]
                for half in range(2):
                    base = half * HIST - _off(j)
                    acc = buf[base, :]

                    def row_body(it, acc, _base=base, _buf=buf):
                        r = _base + 1 + it * 7
                        for u in range(7):
                            acc += _buf[r + u, :]
                        return acc

                    acc = lax.fori_loop(0, 7, row_body, acc)
                    sums_v[(g * K + j) * 2 + half, :] = acc + bias

        fire_group(0, 0)

        def outer(i, _):
            g0 = 2 * i
            fire_group(g0 + 1, 1)
            drain_group(0)
            sum_group(g0, 0)
            g2 = lax.rem(g0 + 2, GROUPS)
            fire_group(g2, 0)
            drain_group(1)
            sum_group(g0 + 1, 1)
            return 0

        lax.fori_loop(0, GROUPS // 2, outer, 0)
        drain_group(0)
        pltpu.sync_copy(sums_v, out_hbm.at[pl.ds(wid * BAGS_PER_W, BAGS_PER_W)])

    return bagsum


def kernel(text, emb_weight, fc1_w, fc1_b, fc2_w, fc2_b):
    et = emb_weight.T
    et_tail = lax.slice(et, (0, MAIN), (EMBED_DIM, VOCAB))
    w2t = jnp.pad(fc2_w.T, ((0, 0), (0, CPAD - NCLS)))
    b2p = jnp.pad(fc2_b, (0, CPAD - NCLS)).reshape(1, CPAD)
    p = _project(et, et_tail, fc1_w.T, w2t, fc1_b.reshape(1, 32), b2p)
    table16 = p.reshape(GROWS, CPAD)

    v = text.astype(jnp.int32)
    linrow = jnp.where(
        v < MAIN,
        (v & ~(STEP - 1)) | ((v & (R - 1)) << 3) | ((v >> 10) & (NK - 1)),
        MAIN + ((v - MAIN) << 3),
    )
    idx = linrow.reshape(-1)

    sums = _make_bagsum()(idx, table16)
    return sums[:, :NCLS]


# speedup vs baseline: 1.2388x; 1.1649x over previous
import functools

import jax
import jax.numpy as jnp
from jax import lax
from jax.experimental import pallas as pl
from jax.experimental.pallas import tpu as pltpu
from jax.experimental.pallas import tpu_sc as plsc

VOCAB = 1000000
EMBED_DIM = 64
NCLS = 10
CPAD = 16
BATCH = 16384
HIST = 50
PAIR = 2 * HIST + 4

R = 2048
NK = 8
STEP = NK * R
NSTEPS = VOCAB // STEP
MAIN = NSTEPS * STEP
TAIL = VOCAB - MAIN
OUTROWS = NSTEPS * R + TAIL + 1
BIASROW = (NSTEPS * R + TAIL) * 8
GROWS = OUTROWS * 8

NUM_WORKERS = 32
BAGS_PER_W = BATCH // NUM_WORKERS
PAIRS_PER_W = BAGS_PER_W // 2
IDX_PER_W = PAIRS_PER_W * 2 * HIST
K = 8
GROUPS = PAIRS_PER_W // K


def _proj_body(*refs):
    et_refs = refs[:NK]
    et_tail_ref, w1t_ref, w2t_ref, b1_ref, b2_ref, o_ref = refs[NK:]
    i = pl.program_id(0)
    wt = lax.dot_general(w2t_ref[...], w1t_ref[...], (((0,), (1,)), ((), ())),
                         preferred_element_type=jnp.float32) * (1.0 / HIST)

    @pl.when(i < NSTEPS)
    def _main():
        ps = []
        for k in range(NK):
            ps.append(jnp.dot(wt, et_refs[k][...], preferred_element_type=jnp.float32))
        o_ref[...] = jnp.concatenate(ps, axis=0).T

    @pl.when(i == NSTEPS)
    def _tail():
        pt = jnp.dot(wt, et_tail_ref[...], preferred_element_type=jnp.float32)
        o_ref[0:TAIL, :] = jnp.concatenate([pt] * NK, axis=0).T
        o_ref[TAIL:TAIL + 1, 0:CPAD] = (
            jnp.dot(b1_ref[...], w2t_ref[...], preferred_element_type=jnp.float32)
            + b2_ref[...]
        )


def _project(et, et_tail, w1t, w2t, b1, b2):
    ins = [pl.BlockSpec((EMBED_DIM, R), (lambda k: (lambda i: (0, jnp.minimum(i * NK + k, NSTEPS * NK - 1))))(k))
           for k in range(NK)]
    ins += [pl.BlockSpec((EMBED_DIM, TAIL), lambda i: (0, 0)),
            pl.BlockSpec((EMBED_DIM, 32), lambda i: (0, 0)),
            pl.BlockSpec((32, CPAD), lambda i: (0, 0)),
            pl.BlockSpec((1, 32), lambda i: (0, 0)),
            pl.BlockSpec((1, CPAD), lambda i: (0, 0))]
    return pl.pallas_call(
        _proj_body, grid=(NSTEPS + 1,), in_specs=ins,
        out_specs=pl.BlockSpec((R, NK * CPAD), lambda i: (i, 0)),
        out_shape=jax.ShapeDtypeStruct((OUTROWS, NK * CPAD), jnp.float32),
    )(*([et] * NK), et_tail, w1t, w2t, b1, b2)


def _make_bagsum():
    mesh = plsc.VectorSubcoreMesh(core_axis_name="c", subcore_axis_name="s")

    @functools.partial(
        pl.kernel,
        mesh=mesh,
        out_type=jax.ShapeDtypeStruct((BATCH, CPAD), jnp.float32),
        scratch_types=[
            pltpu.VMEM((IDX_PER_W,), jnp.int32),
            pltpu.VMEM((2, K, PAIR, CPAD), jnp.float32),
            pltpu.VMEM((BAGS_PER_W, CPAD), jnp.float32),
            pltpu.VMEM((8, CPAD), jnp.float32),
            pltpu.SemaphoreType.DMA,
        ],
        compiler_params=pltpu.CompilerParams(use_tc_tiling_on_sc=False),
    )
    def bagsum(idx_hbm, table_hbm, out_hbm, idx_v, rows_v, sums_v, bias_v, sem):
        num_cores = jax.lax.axis_size("c")
        wid = lax.axis_index("s") * num_cores + lax.axis_index("c")
        pltpu.sync_copy(idx_hbm.at[pl.ds(wid * IDX_PER_W, IDX_PER_W)], idx_v)
        pltpu.sync_copy(table_hbm.at[pl.ds(BIASROW, 8)], bias_v)

        def _n(j):
            return 2 * HIST if j % 2 == 0 else 2 * HIST + 4

        def _off(j):
            return 0 if j % 2 == 0 else -4

        def fire_group(g, p):
            go = pl.multiple_of(g * (K * 2 * HIST), 8)
            for j in range(K):
                pltpu.async_copy(
                    table_hbm.at[idx_v.at[pl.ds(go + (2 * HIST * j + _off(j)), _n(j))]],
                    rows_v.at[p, j, pl.ds(0, _n(j))], sem,
                )

        def drain_group(p):
            for j in range(K):
                pltpu.make_async_copy(
                    table_hbm.at[idx_v.at[pl.ds(0, _n(j))]],
                    rows_v.at[p, j, pl.ds(0, _n(j))], sem,
                ).wait()

        def sum_group(g, p):
            bias = bias_v[0, :]
            for j in range(K):
                buf = rows_v.at[p, j]
                for half in range(2):
                    base = half * HIST - _off(j)
                    acc = buf[base, :]

                    def row_body(it, acc, _base=base, _buf=buf):
                        r = _base + 1 + it * 7
                        for u in range(7):
                            acc += _buf[r + u, :]
                        return acc

                    acc = lax.fori_loop(0, 7, row_body, acc)
                    sums_v[(g * K + j) * 2 + half, :] = acc + bias

        fire_group(0, 0)

        def outer(i, _):
            g0 = 2 * i
            fire_group(g0 + 1, 1)
            drain_group(0)
            sum_group(g0, 0)
            g2 = lax.rem(g0 + 2, GROUPS)
            fire_group(g2, 0)
            drain_group(1)
            sum_group(g0 + 1, 1)
            return 0

        lax.fori_loop(0, GROUPS // 2, outer, 0)
        drain_group(0)
        pltpu.sync_copy(sums_v, out_hbm.at[pl.ds(wid * BAGS_PER_W, BAGS_PER_W)])

    return bagsum


def kernel(text, emb_weight, fc1_w, fc1_b, fc2_w, fc2_b):
    et = emb_weight.T
    et_tail = lax.slice(et, (0, MAIN), (EMBED_DIM, VOCAB))
    w2t = jnp.pad(fc2_w.T, ((0, 0), (0, CPAD - NCLS)))
    b2p = jnp.pad(fc2_b, (0, CPAD - NCLS)).reshape(1, CPAD)
    p = _project(et, et_tail, fc1_w.T, w2t, fc1_b.reshape(1, 32), b2p)
    table16 = p.reshape(GROWS, CPAD)

    v = text.astype(jnp.int32)
    linrow = jnp.where(
        v < MAIN,
        (v & ~(STEP - 1)) | ((v & (R - 1)) << 3) | ((v >> 11) & (NK - 1)),
        MAIN + ((v - MAIN) << 3),
    )
    idx = linrow.reshape(-1)

    sums = _make_bagsum()(idx, table16)
    return sums[:, :NCLS]


# speedup vs baseline: 1.3143x; 1.0610x over previous
import functools

import jax
import jax.numpy as jnp
from jax import lax
from jax.experimental import pallas as pl
from jax.experimental.pallas import tpu as pltpu
from jax.experimental.pallas import tpu_sc as plsc

VOCAB = 1000000
EMBED_DIM = 64
NCLS = 10
CPAD = 16
BATCH = 16384
HIST = 50
PAIR = 2 * HIST + 4

R = 4096
NK = 8
STEP = NK * R
NSTEPS = VOCAB // STEP
MAIN = NSTEPS * STEP
TAILV = VOCAB - MAIN
NTAILSTEPS = -(-(TAILV + 1) // R)
LASTROW = TAILV - (NTAILSTEPS - 1) * R
OUTROWS = NSTEPS * R + TAILV + 1
BIASROW = (NSTEPS * R + TAILV) * 8
GROWS = OUTROWS * 8

NUM_WORKERS = 32
BAGS_PER_W = BATCH // NUM_WORKERS
PAIRS_PER_W = BAGS_PER_W // 2
IDX_PER_W = PAIRS_PER_W * 2 * HIST
K = 8
GROUPS = PAIRS_PER_W // K


def _proj_body(*refs):
    et_refs = refs[:NK]
    w1t_ref, w2t_ref, b1_ref, b2_ref, o_ref = refs[NK:]
    i = pl.program_id(0)
    wt = lax.dot_general(w2t_ref[...], w1t_ref[...], (((0,), (1,)), ((), ())),
                         preferred_element_type=jnp.float32) * (1.0 / HIST)

    @pl.when(i < NSTEPS)
    def _main():
        ps = []
        for k in range(NK):
            ps.append(jnp.dot(wt, et_refs[k][...], preferred_element_type=jnp.float32))
        o_ref[...] = jnp.concatenate(ps, axis=0).T

    @pl.when(i >= NSTEPS)
    def _tail():
        pt = jnp.dot(wt, et_refs[0][...], preferred_element_type=jnp.float32)
        o_ref[...] = jnp.concatenate([pt] * NK, axis=0).T

    @pl.when(i == NSTEPS + NTAILSTEPS - 1)
    def _bias():
        o_ref[LASTROW:LASTROW + 1, 0:CPAD] = (
            jnp.dot(b1_ref[...], w2t_ref[...], preferred_element_type=jnp.float32)
            + b2_ref[...]
        )


def _project(et, w1t, w2t, b1, b2):
    def _map0(i):
        return (0, jnp.where(i < NSTEPS, i * NK, NSTEPS * NK + (i - NSTEPS)))

    ins = [pl.BlockSpec((EMBED_DIM, R), _map0)]
    ins += [pl.BlockSpec((EMBED_DIM, R), (lambda k: (lambda i: (0, jnp.minimum(i, NSTEPS - 1) * NK + k)))(k))
            for k in range(1, NK)]
    ins += [pl.BlockSpec((EMBED_DIM, 32), lambda i: (0, 0)),
            pl.BlockSpec((32, CPAD), lambda i: (0, 0)),
            pl.BlockSpec((1, 32), lambda i: (0, 0)),
            pl.BlockSpec((1, CPAD), lambda i: (0, 0))]
    return pl.pallas_call(
        _proj_body, grid=(NSTEPS + NTAILSTEPS,), in_specs=ins,
        out_specs=pl.BlockSpec((R, NK * CPAD), lambda i: (i, 0)),
        out_shape=jax.ShapeDtypeStruct((OUTROWS, NK * CPAD), jnp.float32),
    )(*([et] * NK), w1t, w2t, b1, b2)


def _make_bagsum():
    mesh = plsc.VectorSubcoreMesh(core_axis_name="c", subcore_axis_name="s")

    @functools.partial(
        pl.kernel,
        mesh=mesh,
        out_type=jax.ShapeDtypeStruct((BATCH, CPAD), jnp.float32),
        scratch_types=[
            pltpu.VMEM((IDX_PER_W,), jnp.int32),
            pltpu.VMEM((2, K, PAIR, CPAD), jnp.float32),
            pltpu.VMEM((BAGS_PER_W, CPAD), jnp.float32),
            pltpu.VMEM((8, CPAD), jnp.float32),
            pltpu.SemaphoreType.DMA,
        ],
        compiler_params=pltpu.CompilerParams(use_tc_tiling_on_sc=False),
    )
    def bagsum(idx_hbm, table_hbm, out_hbm, idx_v, rows_v, sums_v, bias_v, sem):
        num_cores = jax.lax.axis_size("c")
        wid = lax.axis_index("s") * num_cores + lax.axis_index("c")
        pltpu.sync_copy(idx_hbm.at[pl.ds(wid * IDX_PER_W, IDX_PER_W)], idx_v)
        pltpu.sync_copy(table_hbm.at[pl.ds(BIASROW, 8)], bias_v)

        def _n(j):
            return 2 * HIST if j % 2 == 0 else 2 * HIST + 4

        def _off(j):
            return 0 if j % 2 == 0 else -4

        def fire_group(g, p):
            go = pl.multiple_of(g * (K * 2 * HIST), 8)
            for j in range(K):
                pltpu.async_copy(
                    table_hbm.at[idx_v.at[pl.ds(go + (2 * HIST * j + _off(j)), _n(j))]],
                    rows_v.at[p, j, pl.ds(0, _n(j))], sem,
                )

        def drain_group(p):
            for j in range(K):
                pltpu.make_async_copy(
                    table_hbm.at[idx_v.at[pl.ds(0, _n(j))]],
                    rows_v.at[p, j, pl.ds(0, _n(j))], sem,
                ).wait()

        def sum_group(g, p):
            bias = bias_v[0, :]
            for j in range(K):
                buf = rows_v.at[p, j]
                for half in range(2):
                    base = half * HIST - _off(j)
                    acc = buf[base, :]

                    def row_body(it, acc, _base=base, _buf=buf):
                        r = _base + 1 + it * 7
                        for u in range(7):
                            acc += _buf[r + u, :]
                        return acc

                    acc = lax.fori_loop(0, 7, row_body, acc)
                    sums_v[(g * K + j) * 2 + half, :] = acc + bias

        fire_group(0, 0)

        def outer(i, _):
            g0 = 2 * i
            fire_group(g0 + 1, 1)
            drain_group(0)
            sum_group(g0, 0)
            g2 = lax.rem(g0 + 2, GROUPS)
            fire_group(g2, 0)
            drain_group(1)
            sum_group(g0 + 1, 1)
            return 0

        lax.fori_loop(0, GROUPS // 2, outer, 0)
        drain_group(0)
        pltpu.sync_copy(sums_v, out_hbm.at[pl.ds(wid * BAGS_PER_W, BAGS_PER_W)])

    return bagsum


def kernel(text, emb_weight, fc1_w, fc1_b, fc2_w, fc2_b):
    et = emb_weight.T
    w2t = jnp.pad(fc2_w.T, ((0, 0), (0, CPAD - NCLS)))
    b2p = jnp.pad(fc2_b, (0, CPAD - NCLS)).reshape(1, CPAD)
    p = _project(et, fc1_w.T, w2t, fc1_b.reshape(1, 32), b2p)
    table16 = p.reshape(GROWS, CPAD)

    v = text.astype(jnp.int32)
    linrow = jnp.where(
        v < MAIN,
        (v & ~(STEP - 1)) | ((v & (R - 1)) << 3) | ((v >> 12) & (NK - 1)),
        MAIN + ((v - MAIN) << 3),
    )
    idx = linrow.reshape(-1)

    sums = _make_bagsum()(idx, table16)
    return sums[:, :NCLS]


# speedup vs baseline: 1.3241x; 1.0074x over previous
import functools

import jax
import jax.numpy as jnp
from jax import lax
from jax.experimental import pallas as pl
from jax.experimental.pallas import tpu as pltpu
from jax.experimental.pallas import tpu_sc as plsc

VOCAB = 1000000
EMBED_DIM = 64
NCLS = 10
CPAD = 16
BATCH = 16384
HIST = 50
PAIR = 2 * HIST + 4

R = 8192
NK = 8
STEP = NK * R
NSTEPS = VOCAB // STEP
MAIN = NSTEPS * STEP
TAILV = VOCAB - MAIN
NTAILSTEPS = -(-(TAILV + 1) // R)
LASTROW = TAILV - (NTAILSTEPS - 1) * R
OUTROWS = NSTEPS * R + TAILV + 1
BIASROW = (NSTEPS * R + TAILV) * 8
GROWS = OUTROWS * 8

NUM_WORKERS = 32
BAGS_PER_W = BATCH // NUM_WORKERS
PAIRS_PER_W = BAGS_PER_W // 2
IDX_PER_W = PAIRS_PER_W * 2 * HIST
K = 8
GROUPS = PAIRS_PER_W // K


def _proj_body(*refs):
    et_refs = refs[:NK]
    w1t_ref, w2t_ref, b1_ref, b2_ref, o_ref = refs[NK:]
    i = pl.program_id(0)
    wt = lax.dot_general(w2t_ref[...], w1t_ref[...], (((0,), (1,)), ((), ())),
                         preferred_element_type=jnp.float32) * (1.0 / HIST)

    @pl.when(i < NSTEPS)
    def _main():
        ps = []
        for k in range(NK):
            ps.append(jnp.dot(wt, et_refs[k][...], preferred_element_type=jnp.float32))
        o_ref[...] = jnp.concatenate(ps, axis=0).T

    @pl.when(i >= NSTEPS)
    def _tail():
        pt = jnp.dot(wt, et_refs[0][...], preferred_element_type=jnp.float32)
        o_ref[...] = jnp.concatenate([pt] * NK, axis=0).T

    @pl.when(i == NSTEPS + NTAILSTEPS - 1)
    def _bias():
        o_ref[LASTROW:LASTROW + 1, 0:CPAD] = (
            jnp.dot(b1_ref[...], w2t_ref[...], preferred_element_type=jnp.float32)
            + b2_ref[...]
        )


def _project(et, w1t, w2t, b1, b2):
    def _map0(i):
        return (0, jnp.where(i < NSTEPS, i * NK, NSTEPS * NK + (i - NSTEPS)))

    ins = [pl.BlockSpec((EMBED_DIM, R), _map0)]
    ins += [pl.BlockSpec((EMBED_DIM, R), (lambda k: (lambda i: (0, jnp.minimum(i, NSTEPS - 1) * NK + k)))(k))
            for k in range(1, NK)]
    ins += [pl.BlockSpec((EMBED_DIM, 32), lambda i: (0, 0)),
            pl.BlockSpec((32, CPAD), lambda i: (0, 0)),
            pl.BlockSpec((1, 32), lambda i: (0, 0)),
            pl.BlockSpec((1, CPAD), lambda i: (0, 0))]
    return pl.pallas_call(
        _proj_body, grid=(NSTEPS + NTAILSTEPS,), in_specs=ins,
        out_specs=pl.BlockSpec((R, NK * CPAD), lambda i: (i, 0)),
        out_shape=jax.ShapeDtypeStruct((OUTROWS, NK * CPAD), jnp.float32),
    )(*([et] * NK), w1t, w2t, b1, b2)


def _make_bagsum():
    mesh = plsc.VectorSubcoreMesh(core_axis_name="c", subcore_axis_name="s")

    @functools.partial(
        pl.kernel,
        mesh=mesh,
        out_type=jax.ShapeDtypeStruct((BATCH, CPAD), jnp.float32),
        scratch_types=[
            pltpu.VMEM((IDX_PER_W,), jnp.int32),
            pltpu.VMEM((2, K, PAIR, CPAD), jnp.float32),
            pltpu.VMEM((BAGS_PER_W, CPAD), jnp.float32),
            pltpu.VMEM((8, CPAD), jnp.float32),
            pltpu.SemaphoreType.DMA,
        ],
        compiler_params=pltpu.CompilerParams(use_tc_tiling_on_sc=False),
    )
    def bagsum(idx_hbm, table_hbm, out_hbm, idx_v, rows_v, sums_v, bias_v, sem):
        num_cores = jax.lax.axis_size("c")
        wid = lax.axis_index("s") * num_cores + lax.axis_index("c")
        pltpu.sync_copy(idx_hbm.at[pl.ds(wid * IDX_PER_W, IDX_PER_W)], idx_v)
        pltpu.sync_copy(table_hbm.at[pl.ds(BIASROW, 8)], bias_v)

        def _n(j):
            return 2 * HIST if j % 2 == 0 else 2 * HIST + 4

        def _off(j):
            return 0 if j % 2 == 0 else -4

        def fire_group(g, p):
            go = pl.multiple_of(g * (K * 2 * HIST), 8)
            for j in range(K):
                pltpu.async_copy(
                    table_hbm.at[idx_v.at[pl.ds(go + (2 * HIST * j + _off(j)), _n(j))]],
                    rows_v.at[p, j, pl.ds(0, _n(j))], sem,
                )

        def drain_group(p):
            for j in range(K):
                pltpu.make_async_copy(
                    table_hbm.at[idx_v.at[pl.ds(0, _n(j))]],
                    rows_v.at[p, j, pl.ds(0, _n(j))], sem,
                ).wait()

        def sum_group(g, p):
            bias = bias_v[0, :]
            for j in range(K):
                buf = rows_v.at[p, j]
                for half in range(2):
                    base = half * HIST - _off(j)
                    acc = buf[base, :]

                    def row_body(it, acc, _base=base, _buf=buf):
                        r = _base + 1 + it * 7
                        for u in range(7):
                            acc += _buf[r + u, :]
                        return acc

                    acc = lax.fori_loop(0, 7, row_body, acc)
                    sums_v[(g * K + j) * 2 + half, :] = acc + bias

        fire_group(0, 0)

        def outer(i, _):
            g0 = 2 * i
            fire_group(g0 + 1, 1)
            drain_group(0)
            sum_group(g0, 0)
            g2 = lax.rem(g0 + 2, GROUPS)
            fire_group(g2, 0)
            drain_group(1)
            sum_group(g0 + 1, 1)
            return 0

        lax.fori_loop(0, GROUPS // 2, outer, 0)
        drain_group(0)
        pltpu.sync_copy(sums_v, out_hbm.at[pl.ds(wid * BAGS_PER_W, BAGS_PER_W)])

    return bagsum


def kernel(text, emb_weight, fc1_w, fc1_b, fc2_w, fc2_b):
    et = emb_weight.T
    w2t = jnp.pad(fc2_w.T, ((0, 0), (0, CPAD - NCLS)))
    b2p = jnp.pad(fc2_b, (0, CPAD - NCLS)).reshape(1, CPAD)
    p = _project(et, fc1_w.T, w2t, fc1_b.reshape(1, 32), b2p)
    table16 = p.reshape(GROWS, CPAD)

    v = text.astype(jnp.int32)
    linrow = jnp.where(
        v < MAIN,
        (v & ~(STEP - 1)) | ((v & (R - 1)) << 3) | ((v >> 13) & (NK - 1)),
        MAIN + ((v - MAIN) << 3),
    )
    idx = linrow.reshape(-1)

    sums = _make_bagsum()(idx, table16)
    return sums[:, :NCLS]
